# baseline (speedup 1.0000x reference)
import jax
import jax.numpy as jnp
from jax.experimental import pallas as pl
from jax.experimental.pallas import tpu as pltpu

TS = 1024
NX = 4


def _adapter_body(*refs):
    x_refs = refs[:NX]
    dw_refs = refs[NX:2 * NX]
    db_ref, uw_ref, o_ref = refs[2 * NX:]
    db = db_ref[0, 0, 0]
    uw = uw_ref[0, 0]
    z = db[None, :].astype(jnp.float32)
    for xr, dr in zip(x_refs, dw_refs):
        z = z + jnp.dot(xr[0], dr[0, 0], preferred_element_type=jnp.float32)
    z = z * jax.nn.sigmoid(z)
    o_ref[0, 0] = jnp.dot(z, uw, preferred_element_type=jnp.float32)


@jax.jit
def kernel(x, expert_index, down_w, down_b, up_w):
    B, S, C = x.shape
    M, N, _, D = down_w.shape
    CH = C // NX
    s_blocks = S // TS

    idx = expert_index.astype(jnp.int32)
    m = jnp.arange(M)[:, None]
    bdw = down_w[m, idx]
    bdb = down_b[m, idx].reshape(M, B, 1, D)
    buw = up_w[m, idx]

    grid = (M, B, s_blocks)

    x_specs = [
        pl.BlockSpec((1, TS, CH), (lambda q: lambda mm, b, s: (b, s, q))(q))
        for q in range(NX)
    ]
    dw_specs = [
        pl.BlockSpec((1, 1, CH, D), (lambda q: lambda mm, b, s: (mm, b, q, 0))(q))
        for q in range(NX)
    ]

    out = pl.pallas_call(
        _adapter_body,
        grid=grid,
        in_specs=x_specs + dw_specs + [
            pl.BlockSpec((1, 1, 1, D), lambda mm, b, s: (mm, b, 0, 0)),
            pl.BlockSpec((1, 1, D, C), lambda mm, b, s: (mm, b, 0, 0)),
        ],
        out_specs=pl.BlockSpec((1, 1, TS, C), lambda mm, b, s: (mm, b, s, 0)),
        out_shape=jax.ShapeDtypeStruct((M, B, S, C), jnp.float32),
        compiler_params=pltpu.CompilerParams(
            dimension_semantics=("parallel", "parallel", "parallel"),
            vmem_limit_bytes=120 * 1024 * 1024,
        ),
    )(*([x] * NX), *([bdw] * NX), bdb, buw)
    return out
